# Initial kernel scaffold; baseline (speedup 1.0000x reference)
import jax
import jax.numpy as jnp
from jax.experimental import pallas as pl


def kernel(inputs):
    raise NotImplementedError("write your pallas kernel here")


# speedup vs baseline: 30.6710x; 30.6710x over previous
import jax
import jax.numpy as jnp
from jax.experimental import pallas as pl

_K = 8
_NEG = float("-inf")


def _body(x_ref, o_ref):
    x = x_ref[0]
    bound = jnp.full((1, x.shape[1]), jnp.inf, jnp.float32)
    vals = []
    cnts = []
    for _ in range(_K):
        y = jnp.where(x < bound, x, _NEG)
        m = jnp.max(y, axis=0, keepdims=True)
        c = jnp.sum((y == m).astype(jnp.int32), axis=0, keepdims=True)
        vals.append(m)
        cnts.append(c)
        bound = m
    q = []
    acc = jnp.zeros_like(cnts[0])
    for c in cnts:
        acc = acc + c
        q.append(acc)
    rows = []
    for s in range(_K):
        r = vals[_K - 1]
        for j in range(_K - 2, -1, -1):
            r = jnp.where(s < q[j], vals[j], r)
        rows.append(r)
    o_ref[0] = jnp.concatenate(rows, axis=0)


def kernel(inputs):
    B, S, C = inputs.shape
    out = pl.pallas_call(
        _body,
        grid=(B,),
        in_specs=[pl.BlockSpec((1, S, C), lambda b: (b, 0, 0))],
        out_specs=pl.BlockSpec((1, _K, C), lambda b: (b, 0, 0)),
        out_shape=jax.ShapeDtypeStruct((B, _K, C), jnp.float32),
    )(inputs)
    return jnp.transpose(out, (0, 2, 1))
